# Initial kernel scaffold; baseline (speedup 1.0000x reference)
import jax
import jax.numpy as jnp
from jax.experimental import pallas as pl


def kernel(x, edge_attr, edge_feature, edge_index, batch, conv1_We, conv1_be, conv1_W1, conv1_b1, conv1_W2, conv1_b2, conv2_We, conv2_be, conv2_W1, conv2_b1, conv2_W2, conv2_b2, lstm_Wih, lstm_Whh, lstm_bih, lstm_bhh, dense_W, dense_b, out_W, out_b):
    raise NotImplementedError("write your pallas kernel here")


# speedup vs baseline: 3.9187x; 3.9187x over previous
import functools

import jax
import jax.numpy as jnp
from jax import lax
from jax.experimental import pallas as pl
from jax.experimental.pallas import tpu as pltpu
from jax.experimental.pallas import tpu_sc as plsc

_N, _E, _D, _DE, _H, _G, _STEPS = 10000, 320000, 128, 16, 64, 32, 3
_NC, _NS = 2, 16
_NW = _NC * _NS
_CH = 64
_NP = 10240


def _edge_embed_body(a_ref, b_ref, w1a_ref, w1b_ref, be1_ref, w2a_ref,
                     w2b_ref, be2_ref, e1_ref, e2_ref):
    a = a_ref[...]
    b = b_ref[...]
    e1_ref[...] = (
        jnp.dot(a, w1a_ref[...], preferred_element_type=jnp.float32)
        + jnp.dot(b, w1b_ref[...], preferred_element_type=jnp.float32)
        + be1_ref[...])
    e2_ref[...] = (
        jnp.dot(a, w2a_ref[...], preferred_element_type=jnp.float32)
        + jnp.dot(b, w2b_ref[...], preferred_element_type=jnp.float32)
        + be2_ref[...])


def _edge_embed(ea_a, ea_b, w1a, w1b, be1, w2a, w2b, be2):
    blk = 1600
    grid = _E // blk
    return pl.pallas_call(
        _edge_embed_body,
        grid=(grid,),
        in_specs=[
            pl.BlockSpec((blk, _DE), lambda i: (i, 0)),
            pl.BlockSpec((blk, _DE), lambda i: (i, 0)),
            pl.BlockSpec((_DE, _D), lambda i: (0, 0)),
            pl.BlockSpec((_DE, _D), lambda i: (0, 0)),
            pl.BlockSpec((1, _D), lambda i: (0, 0)),
            pl.BlockSpec((_DE, _D), lambda i: (0, 0)),
            pl.BlockSpec((_DE, _D), lambda i: (0, 0)),
            pl.BlockSpec((1, _D), lambda i: (0, 0)),
        ],
        out_specs=[
            pl.BlockSpec((blk, _D), lambda i: (i, 0)),
            pl.BlockSpec((blk, _D), lambda i: (i, 0)),
        ],
        out_shape=[
            jax.ShapeDtypeStruct((_E, _D), jnp.float32),
            jax.ShapeDtypeStruct((_E, _D), jnp.float32),
        ],
    )(ea_a, ea_b, w1a, w1b, be1, w2a, w2b, be2)


def _make_edge_sc(dd):
    n_chunks = _E // _CH
    base = n_chunks // _NW
    extra = n_chunks % _NW
    rows_per_tile = _NP // _NS
    n_full = rows_per_tile // _CH
    nlane = dd // 16

    mesh = plsc.VectorSubcoreMesh(core_axis_name="c", subcore_axis_name="s")

    @functools.partial(
        pl.kernel,
        out_type=jax.ShapeDtypeStruct((2 * _NP, dd), jnp.float32),
        mesh=mesh,
        scratch_types=[
            pltpu.VMEM((_CH,), jnp.int32),
            pltpu.VMEM((_CH,), jnp.int32),
            pltpu.VMEM((_CH,), jnp.int32),
            pltpu.VMEM((_CH,), jnp.int32),
            pltpu.VMEM((_CH, _D), jnp.float32),
            pltpu.VMEM((_CH, _D), jnp.float32),
            pltpu.VMEM((_CH, dd), jnp.float32),
            pltpu.VMEM((_CH, dd), jnp.float32),
            pltpu.VMEM_SHARED((_NP, dd), jnp.float32),
            pltpu.SemaphoreType.DMA,
            pltpu.SemaphoreType.DMA,
            pltpu.SemaphoreType.DMA,
            pltpu.SemaphoreType.DMA,
            pltpu.SemaphoreType.DMA,
            pltpu.SemaphoreType.DMA,
        ],
    )
    def k(x_hbm, src_hbm, dst_hbm, e_hbm, out_hbm,
          src0, src1, dst0, dst1, xg0, xg1, ev0, ev1, acc_sh,
          si0, si1, sg0, sg1, se0, se1):
        cid = lax.axis_index("c")
        sid = lax.axis_index("s")
        w = cid * _NS + sid
        srcs = (src0, src1)
        dsts = (dst0, dst1)
        xgs = (xg0, xg1)
        evs = (ev0, ev1)
        sis = (si0, si1)
        sgs = (sg0, sg1)
        ses = (se0, se1)

        def zrow(i, _):
            for j in range(nlane):
                ev0[i, pl.ds(j * 16, 16)] = jnp.zeros((16,), jnp.float32)
            return 0
        lax.fori_loop(0, _CH, zrow, 0)

        r0 = sid * rows_per_tile
        for t in range(n_full):
            pltpu.sync_copy(ev0, acc_sh.at[pl.ds(r0 + t * _CH, _CH)])
        plsc.subcore_barrier()

        c_start = w * base + jnp.minimum(w, extra)
        n_my = base + jnp.where(w < extra, 1, 0)

        def ebase(c):
            return (c_start + c) * _CH

        def issue_idx(c, b):
            pltpu.async_copy(src_hbm.at[pl.ds(ebase(c), _CH)], srcs[b],
                             sis[b])
            pltpu.async_copy(dst_hbm.at[pl.ds(ebase(c), _CH)], dsts[b],
                             sis[b])

        def wait_idx(b):
            pltpu.make_async_copy(src_hbm.at[pl.ds(0, _CH)], srcs[b],
                                  sis[b]).wait()
            pltpu.make_async_copy(dst_hbm.at[pl.ds(0, _CH)], dsts[b],
                                  sis[b]).wait()

        def issue_ge(c, b):
            pltpu.async_copy(x_hbm.at[srcs[b]], xgs[b], sgs[b])
            pltpu.async_copy(e_hbm.at[pl.ds(ebase(c), _CH)], evs[b], ses[b])

        def wait_ge(b):
            pltpu.make_async_copy(x_hbm.at[pl.ds(0, _CH)], xgs[b],
                                  sgs[b]).wait()
            pltpu.make_async_copy(e_hbm.at[pl.ds(0, _CH)], evs[b],
                                  ses[b]).wait()

        pltpu.sync_copy(src_hbm.at[pl.ds(ebase(0), _CH)], src0)
        pltpu.sync_copy(dst_hbm.at[pl.ds(ebase(0), _CH)], dst0)
        pltpu.async_copy(x_hbm.at[src0], xg0, sg0)
        pltpu.async_copy(e_hbm.at[pl.ds(ebase(0), _CH)], ev0, se0)

        @pl.when(1 < n_my)
        def _():
            issue_idx(1, 1)

        def pair_body(ip, _):
            for b in range(2):
                c = 2 * ip + b
                nb = 1 - b

                @pl.when(c < n_my)
                def _():
                    @pl.when(c + 1 < n_my)
                    def _():
                        wait_idx(nb)
                        issue_ge(c + 1, nb)

                    wait_ge(b)

                    def crow(r, _):
                        for j in range(nlane):
                            sl = pl.ds(j * 16, 16)
                            evs[b][r, sl] = jnp.maximum(
                                xgs[b][r, sl] + evs[b][r, sl], 0.0)
                        return 0
                    lax.fori_loop(0, _CH, crow, 0)

                    pltpu.sync_copy(evs[b], acc_sh.at[dsts[b]], add=True)

                    @pl.when(c + 2 < n_my)
                    def _():
                        issue_idx(c + 2, b)
            return 0
        lax.fori_loop(0, (base + 2) // 2, pair_body, 0)

        plsc.subcore_barrier()

        out_base = cid * _NP + r0
        for t in range(n_full):
            pltpu.sync_copy(acc_sh.at[pl.ds(r0 + t * _CH, _CH)], ev0)
            pltpu.sync_copy(ev0, out_hbm.at[pl.ds(out_base + t * _CH, _CH)])

    return k


_make_edge_sc_cached = functools.cache(_make_edge_sc)


def _edge_sc(dd, x, src, dst, e):
    return _make_edge_sc_cached(dd)(x, src, dst, e)


def _node_mlp_body(x_ref, p_ref, w1_ref, b1_ref, w2_ref, b2_ref, out_ref):
    h = x_ref[...] + p_ref[0, :_N, :] + p_ref[1, :_N, :]
    t = jnp.maximum(
        jnp.dot(h, w1_ref[...], preferred_element_type=jnp.float32)
        + b1_ref[...], 0.0)
    out_ref[...] = jnp.maximum(
        jnp.dot(t, w2_ref[...], preferred_element_type=jnp.float32)
        + b2_ref[...], 0.0)


def _node_mlp(x, p, w1, b1, w2, b2):
    return pl.pallas_call(
        _node_mlp_body,
        out_shape=jax.ShapeDtypeStruct((_N, _D), jnp.float32),
    )(x, p, w1, b1, w2, b2)


def _final_body(hh_ref, p_ref, w1_ref, b1_ref, w2_ref, b2_ref, batch_ref,
                wih_ref, whh_ref, bih_ref, bhh_ref, dw_ref, db_ref,
                ow_ref, ob_ref, out_ref):
    h2 = hh_ref[:, :_H] + p_ref[0, :_N, :_H] + p_ref[1, :_N, :_H]
    t = jnp.maximum(
        jnp.dot(h2, w1_ref[...], preferred_element_type=jnp.float32)
        + b1_ref[...], 0.0)
    xs = jnp.maximum(
        jnp.dot(t, w2_ref[...], preferred_element_type=jnp.float32)
        + b2_ref[...], 0.0)

    batch = batch_ref[...]
    gids = lax.broadcasted_iota(jnp.int32, (_N, _G), 1)
    mask = batch == gids

    wih = wih_ref[...]
    whh = whh_ref[...]
    bih = bih_ref[...]
    bhh = bhh_ref[...]

    q_star = jnp.zeros((_G, 2 * _H), jnp.float32)
    hs = jnp.zeros((_G, _H), jnp.float32)
    cs = jnp.zeros((_G, _H), jnp.float32)
    for _ in range(_STEPS):
        gates = (
            lax.dot_general(q_star, wih, (((1,), (1,)), ((), ())),
                            preferred_element_type=jnp.float32)
            + bih
            + lax.dot_general(hs, whh, (((1,), (1,)), ((), ())),
                              preferred_element_type=jnp.float32)
            + bhh)
        gi = jax.nn.sigmoid(gates[:, :_H])
        gf = jax.nn.sigmoid(gates[:, _H:2 * _H])
        gg = jnp.tanh(gates[:, 2 * _H:3 * _H])
        go = jax.nn.sigmoid(gates[:, 3 * _H:])
        cs = gf * cs + gi * gg
        hs = go * jnp.tanh(cs)

        s = lax.dot_general(xs, hs, (((1,), (1,)), ((), ())),
                            preferred_element_type=jnp.float32)
        sm = jnp.where(mask, s, -jnp.inf)
        emax = jnp.max(sm, axis=0, keepdims=True)
        emax = jnp.where(jnp.isfinite(emax), emax, 0.0)
        a = jnp.where(mask, jnp.exp(s - emax), 0.0)
        denom = jnp.sum(a, axis=0, keepdims=True)
        an = a / (denom + 1e-16)
        r = lax.dot_general(an, xs, (((0,), (0,)), ((), ())),
                            preferred_element_type=jnp.float32)
        q_star = jnp.concatenate([hs, r], axis=1)

    z = jnp.maximum(
        jnp.dot(q_star, dw_ref[...], preferred_element_type=jnp.float32)
        + db_ref[...], 0.0)
    out_ref[...] = (
        jnp.dot(z, ow_ref[...], preferred_element_type=jnp.float32)
        + ob_ref[...])


def _final(hh, p, w1, b1, w2, b2, batch2, wih, whh, bih, bhh, dw, db, ow, ob):
    return pl.pallas_call(
        _final_body,
        out_shape=jax.ShapeDtypeStruct((_G, 1), jnp.float32),
    )(hh, p, w1, b1, w2, b2, batch2, wih, whh, bih, bhh, dw, db, ow, ob)


@jax.jit
def kernel(x, edge_attr, edge_feature, edge_index, batch,
           conv1_We, conv1_be, conv1_W1, conv1_b1, conv1_W2, conv1_b2,
           conv2_We, conv2_be, conv2_W1, conv2_b1, conv2_W2, conv2_b2,
           lstm_Wih, lstm_Whh, lstm_bih, lstm_bhh,
           dense_W, dense_b, out_W, out_b):
    src = edge_index[0]
    dst = edge_index[1]

    w2e = jnp.pad(conv2_We, ((0, 0), (0, _D - _H)))
    b2e = jnp.pad(conv2_be, (0, _D - _H)).reshape(1, _D)
    e1, e2 = _edge_embed(
        edge_attr, edge_feature,
        conv1_We[:_DE], conv1_We[_DE:], conv1_be.reshape(1, _D),
        w2e[:_DE], w2e[_DE:], b2e)

    p1 = _edge_sc(_D, x, src, dst, e1).reshape(2, _NP, _D)
    w2p = jnp.pad(conv1_W2, ((0, 0), (0, _D - _H)))
    b2p = jnp.pad(conv1_b2, (0, _D - _H)).reshape(1, _D)
    hh = _node_mlp(x, p1, conv1_W1, conv1_b1.reshape(1, _H), w2p, b2p)

    p2 = _edge_sc(_D, hh, src, dst, e2).reshape(2, _NP, _D)
    out = _final(hh, p2, conv2_W1, conv2_b1.reshape(1, _H),
                 conv2_W2, conv2_b2.reshape(1, _H),
                 batch.reshape(_N, 1),
                 lstm_Wih, lstm_Whh,
                 lstm_bih.reshape(1, 4 * _H), lstm_bhh.reshape(1, 4 * _H),
                 dense_W, dense_b.reshape(1, _H), out_W, out_b.reshape(1, 1))
    return out.reshape(_G)
